# Initial kernel scaffold; baseline (speedup 1.0000x reference)
import jax
import jax.numpy as jnp
from jax.experimental import pallas as pl


def kernel(context_tokens, time_tokens, loc_tokens, act_tokens, time_table, loc_table, act_table, ctx_tables, W_time, b_time, W_loc, b_loc, W_act, b_act):
    raise NotImplementedError("write your pallas kernel here")


# speedup vs baseline: 2.3317x; 2.3317x over previous
import dataclasses
import functools

import jax
import jax.numpy as jnp
from jax.experimental import pallas as pl
from jax.experimental.pallas import tpu as pltpu
from jax.experimental.pallas import tpu_sc as plsc

_WIN = 128


def _project_table(table, W, b, block_rows):
    M, Dd = table.shape

    def body(t_ref, w_ref, b_ref, o_ref):
        o_ref[...] = jax.lax.dot_general(
            t_ref[...], w_ref[...],
            dimension_numbers=(((1,), (1,)), ((), ())),
            preferred_element_type=jnp.float32,
            precision=jax.lax.Precision.HIGHEST,
        ) + b_ref[...]

    return pl.pallas_call(
        body,
        grid=(M // block_rows,),
        in_specs=[
            pl.BlockSpec((block_rows, Dd), lambda i: (i, 0)),
            pl.BlockSpec((Dd, Dd), lambda i: (0, 0)),
            pl.BlockSpec((1, Dd), lambda i: (0, 0)),
        ],
        out_specs=pl.BlockSpec((block_rows, Dd), lambda i: (i, 0)),
        out_shape=jax.ShapeDtypeStruct((M, Dd), jnp.float32),
    )(table, W, b.reshape(1, Dd))


def _sc_gathers(time_proj, loc_proj, act_proj, ctx_flat, t_idx, l_idx, a_idx, c_idx):
    n = t_idx.shape[1]
    nc = c_idx.shape[1]
    d = time_proj.shape[1]
    mesh = plsc.VectorSubcoreMesh(core_axis_name="c", subcore_axis_name="s")
    out_type = (
        jax.ShapeDtypeStruct((n, d), jnp.float32),
        jax.ShapeDtypeStruct((n, d), jnp.float32),
        jax.ShapeDtypeStruct((n, d), jnp.float32),
        jax.ShapeDtypeStruct((nc, d), jnp.float32),
    )

    cp = pltpu.CompilerParams()
    if "use_tc_tiling_on_sc" in pltpu.CompilerParams.__dataclass_fields__:
        cp = dataclasses.replace(cp, use_tc_tiling_on_sc=False)

    @functools.partial(pl.kernel, out_type=out_type, mesh=mesh,
                       compiler_params=cp)
    def k(tp_h, lp_h, ap_h, cp_h, ti_h, li_h, ai_h, ci_h, to_h, lo_h, ao_h, co_h):
        def body3(ti_v, li_v, ai_v, to_v, lo_v, ao_v):
            pltpu.sync_copy(tp_h.at[ti_v.at[0]], to_v)
            pltpu.sync_copy(lp_h.at[li_v.at[0]], lo_v)
            pltpu.sync_copy(ap_h.at[ai_v.at[0]], ao_v)

        pltpu.emit_pipeline(
            body3,
            grid=(n // _WIN,),
            in_specs=[pl.BlockSpec((1, _WIN), lambda i: (0, i))] * 3,
            out_specs=[pl.BlockSpec((_WIN, d), lambda i: (i, 0))] * 3,
            core_axis_name=("c", "s"),
            dimension_semantics=(pltpu.PARALLEL,),
        )(ti_h, li_h, ai_h, to_h, lo_h, ao_h)

        def bodyc(ci_v, co_v):
            pltpu.sync_copy(cp_h.at[ci_v.at[0]], co_v)

        pltpu.emit_pipeline(
            bodyc,
            grid=(nc // _WIN,),
            in_specs=[pl.BlockSpec((1, _WIN), lambda i: (0, i))],
            out_specs=[pl.BlockSpec((_WIN, d), lambda i: (i, 0))],
            core_axis_name=("c", "s"),
            dimension_semantics=(pltpu.PARALLEL,),
        )(ci_h, co_h)

    return k(time_proj, loc_proj, act_proj, ctx_flat, t_idx, l_idx, a_idx, c_idx)


def kernel(context_tokens, time_tokens, loc_tokens, act_tokens, time_table,
           loc_table, act_table, ctx_tables, W_time, b_time, W_loc, b_loc,
           W_act, b_act):
    B, L = time_tokens.shape
    NF = context_tokens.shape[1]
    ctx_vocab = ctx_tables.shape[1]
    D = time_table.shape[1]

    time_proj = _project_table(time_table, W_time, b_time, time_table.shape[0])
    act_proj = _project_table(act_table, W_act, b_act, 10000)
    loc_proj = _project_table(loc_table, W_loc, b_loc, 8000)

    ctx_flat = ctx_tables.reshape(NF * ctx_vocab, D)
    c_idx = (context_tokens.astype(jnp.int32)
             + jnp.arange(NF, dtype=jnp.int32)[None, :] * ctx_vocab
             ).reshape(1, B * NF)
    t_idx = time_tokens.astype(jnp.int32).reshape(1, B * L)
    l_idx = loc_tokens.astype(jnp.int32).reshape(1, B * L)
    a_idx = act_tokens.astype(jnp.int32).reshape(1, B * L)

    t_out, l_out, a_out, c_out = _sc_gathers(
        time_proj, loc_proj, act_proj, ctx_flat, t_idx, l_idx, a_idx, c_idx)

    return (c_out.reshape(B, NF, D),
            t_out.reshape(B, L, D),
            l_out.reshape(B, L, D),
            a_out.reshape(B, L, D))
